# baseline (speedup 1.0000x reference)
import jax
import jax.numpy as jnp
from jax import lax
from jax.experimental import pallas as pl
from jax.experimental.pallas import tpu as pltpu
from jax.experimental.pallas import tpu_sc as plsc

N = 10000
E = 320000
CH = 128
HW = 64
OUT = 16
G = 64
NEG = 0.2

NP = 10240
NB = 10
RB = NP // NB

NSUB = 16
EB = 96
NSLOT = 4
NBLK = 212
EPT = NBLK * EB
EPAD = EPT * NSUB
STRIPE = 2 * NP // NSUB
DSTRIPE = NP // NSUB


def _prologue_body(x_ref, w_ref, asr_ref, adr_ref, t_ref, h_ref, a_ref, d_ref):
    hf = pl.program_id(0)
    cls = pl.program_id(1)
    h = jnp.dot(x_ref[...], w_ref[...], preferred_element_type=jnp.float32)
    av = jnp.sum(h * asr_ref[...], axis=1)
    dv = jnp.sum(h * adr_ref[...], axis=1)
    scale = jnp.where(cls == 0, 1.0, NEG)
    p = jnp.exp(scale * av)
    ph = p[:, None] * h
    t_ref[...] = jnp.where(hf == 0, ph[:, :HW], ph[:, HW:])
    h_ref[...] = h
    a_ref[...] = av
    d_ref[...] = dv


def _prologue(xp, w, asr, adr):
    return pl.pallas_call(
        _prologue_body,
        grid=(2, 2, NB),
        in_specs=[
            pl.BlockSpec((RB, CH), lambda hf, c, i: (i, 0)),
            pl.BlockSpec((CH, CH), lambda hf, c, i: (0, 0)),
            pl.BlockSpec((1, CH), lambda hf, c, i: (0, 0)),
            pl.BlockSpec((1, CH), lambda hf, c, i: (0, 0)),
        ],
        out_specs=[
            pl.BlockSpec((RB, HW), lambda hf, c, i: ((hf * 2 + c) * NB + i, 0)),
            pl.BlockSpec((RB, CH), lambda hf, c, i: (i, 0)),
            pl.BlockSpec((RB,), lambda hf, c, i: (i,)),
            pl.BlockSpec((RB,), lambda hf, c, i: (i,)),
        ],
        out_shape=[
            jax.ShapeDtypeStruct((4 * NP, HW), jnp.float32),
            jax.ShapeDtypeStruct((NP, CH), jnp.float32),
            jax.ShapeDtypeStruct((NP,), jnp.float32),
            jax.ShapeDtypeStruct((NP,), jnp.float32),
        ],
    )(xp, w, asr, adr)


def _edge_body(t_hbm, ei_hbm, as_hbm, ad_hbm, z2_hbm, z1_hbm,
               acc_hbm, den_hbm,
               as_v, ad_v, ebuf,
               gbuf0, sidx0, didx0, dval0, rows0,
               gbuf1, sidx1, didx1, dval1, rows1,
               gbuf2, sidx2, didx2, dval2, rows2,
               gbuf3, sidx3, didx3, dval3, rows3,
               acc_s, den_s,
               semg0, semg1, semg2, semg3,
               semr0, semr1, semr2, semr3,
               semd0, semd1, semd2, semd3):
    c = lax.axis_index("c")
    s = lax.axis_index("s")
    slots = ((gbuf0, sidx0, didx0, dval0, rows0, semg0, semr0, semd0),
             (gbuf1, sidx1, didx1, dval1, rows1, semg1, semr1, semd1),
             (gbuf2, sidx2, didx2, dval2, rows2, semg2, semr2, semd2),
             (gbuf3, sidx3, didx3, dval3, rows3, semg3, semr3, semd3))

    pltpu.sync_copy(z2_hbm, acc_s.at[pl.ds(s * STRIPE, STRIPE)])
    pltpu.sync_copy(z1_hbm, den_s.at[pl.ds(s * DSTRIPE, DSTRIPE)])
    pltpu.sync_copy(as_hbm, as_v)
    pltpu.sync_copy(ad_hbm, ad_v)
    plsc.subcore_barrier()

    cneg = jnp.broadcast_to(c == 1, (16,))
    trash = jnp.broadcast_to(N + s, (16,))
    goff = jnp.broadcast_to(c * (2 * NP), (16,))
    npvec = jnp.broadcast_to(NP, (16,))
    zvec = jnp.zeros((16,), jnp.int32)
    ascale = jnp.where(c == 1, NEG, 1.0)
    bbase = s * NBLK

    def classify(off, gbuf, sidx, didx, dval):
        def grp(j, carry2):
            s16 = ebuf[pl.ds(off + j * 16, 16)]
            d16 = ebuf[pl.ds(off + EB + j * 16, 16)]
            a16 = plsc.load_gather(as_v, [s16])
            b16 = plsc.load_gather(ad_v, [d16])
            pos = (a16 + b16) > 0.0
            clsoff = jnp.where(pos, zvec, npvec)
            gbuf[pl.ds(j * 16, 16)] = s16 + goff + clsoff
            sidx[pl.ds(j * 16, 16)] = d16 + clsoff
            didx[pl.ds(j * 16, 16)] = jnp.where(pos != cneg, d16, trash)
            dval[pl.ds(j * 16, 16)] = jnp.exp(ascale * a16)
            return carry2

        lax.fori_loop(0, EB // 16, grp, 0)

    def fire_g(slot):
        gbuf, _, _, _, rows, semg, _, _ = slot
        pltpu.async_copy(t_hbm.at[gbuf], rows, semg)

    def fire_s(slot):
        _, sidx, didx, dval, rows, _, semr, semd = slot
        pltpu.async_copy(rows, acc_s.at[sidx], semr, add=True)
        pltpu.async_copy(dval, den_s.at[didx], semd, add=True)

    def wait_g(slot):
        gbuf, _, _, _, rows, semg, _, _ = slot
        pltpu.make_async_copy(t_hbm.at[gbuf], rows, semg).wait()

    def wait_s(slot):
        _, sidx, didx, dval, rows, _, semr, semd = slot
        pltpu.make_async_copy(rows, acc_s.at[sidx], semr).wait()
        pltpu.make_async_copy(dval, den_s.at[didx], semd).wait()

    def quad(i, first):
        base = (bbase + NSLOT * i) * 2 * EB
        pltpu.sync_copy(ei_hbm.at[pl.ds(base, NSLOT * 2 * EB)], ebuf)
        for k in range(NSLOT):
            if not first:
                wait_s(slots[k])
            classify(k * 2 * EB, *slots[k][:4])
            fire_g(slots[k])
        for k in range(NSLOT):
            wait_g(slots[k])
            fire_s(slots[k])

    quad(0, True)
    lax.fori_loop(1, NBLK // NSLOT, lambda i, cy: (quad(i, False), cy)[1], 0)
    for k in range(NSLOT):
        wait_s(slots[k])
    plsc.subcore_barrier()

    pltpu.sync_copy(acc_s.at[pl.ds(s * STRIPE, STRIPE)],
                    acc_hbm.at[pl.ds(c * 2 * NP + s * STRIPE, STRIPE)])
    pltpu.sync_copy(den_s.at[pl.ds(s * DSTRIPE, DSTRIPE)],
                    den_hbm.at[pl.ds(c * NP + s * DSTRIPE, DSTRIPE)])


def _edge_call(*args):
    return pl.kernel(
        _edge_body,
        out_type=[jax.ShapeDtypeStruct((4 * NP, HW), jnp.float32),
                  jax.ShapeDtypeStruct((2 * NP,), jnp.float32)],
        mesh=plsc.VectorSubcoreMesh(core_axis_name="c", subcore_axis_name="s",
                                    num_cores=2, num_subcores=NSUB),
        compiler_params=pltpu.CompilerParams(needs_layout_passes=False,
                                             use_tc_tiling_on_sc=False),
        scratch_types=(
            [pltpu.VMEM((NP,), jnp.float32),
             pltpu.VMEM((NP,), jnp.float32),
             pltpu.VMEM((NSLOT * 2 * EB,), jnp.int32)]
            + NSLOT * [pltpu.VMEM((EB,), jnp.int32),
                       pltpu.VMEM((EB,), jnp.int32),
                       pltpu.VMEM((EB,), jnp.int32),
                       pltpu.VMEM((EB,), jnp.float32),
                       pltpu.VMEM((EB, HW), jnp.float32)]
            + [pltpu.VMEM_SHARED((2 * NP, HW), jnp.float32),
               pltpu.VMEM_SHARED((NP,), jnp.float32)]
            + 3 * NSLOT * [pltpu.SemaphoreType.DMA]
        ),
    )(*args)


def _combine_body(h_ref, a_ref, d_ref, p0_ref, n0_ref, p1_ref, n1_ref,
                  denp_ref, denn_ref, b_ref, o_ref):
    av = a_ref[...]
    dv = d_ref[...]
    q = jnp.exp(dv)
    q2 = jnp.exp(NEG * dv)
    sl = av + dv
    wself = jnp.where(sl > 0.0, jnp.exp(sl), jnp.exp(NEG * sl))
    hv = h_ref[...]
    den = q * denp_ref[...] + q2 * denn_ref[...] + wself
    inv = 1.0 / (den + 1e-16)
    lo = (q[:, None] * p0_ref[...] + q2[:, None] * n0_ref[...]
          + wself[:, None] * hv[:, :HW])
    hi = (q[:, None] * p1_ref[...] + q2[:, None] * n1_ref[...]
          + wself[:, None] * hv[:, HW:])
    bv = b_ref[...]
    o_ref[...] = jnp.maximum(
        jnp.concatenate([lo, hi], axis=1) * inv[:, None] + bv[None, :], 0.0)


def _combine(h, av, dv, acc, den, b):
    return pl.pallas_call(
        _combine_body,
        grid=(NB,),
        in_specs=[
            pl.BlockSpec((RB, CH), lambda i: (i, 0)),
            pl.BlockSpec((RB,), lambda i: (i,)),
            pl.BlockSpec((RB,), lambda i: (i,)),
            pl.BlockSpec((RB, HW), lambda i: (i, 0)),
            pl.BlockSpec((RB, HW), lambda i: (NB + i, 0)),
            pl.BlockSpec((RB, HW), lambda i: (2 * NB + i, 0)),
            pl.BlockSpec((RB, HW), lambda i: (3 * NB + i, 0)),
            pl.BlockSpec((RB,), lambda i: (i,)),
            pl.BlockSpec((RB,), lambda i: (NB + i,)),
            pl.BlockSpec((CH,), lambda i: (0,)),
        ],
        out_specs=pl.BlockSpec((RB, CH), lambda i: (i, 0)),
        out_shape=jax.ShapeDtypeStruct((NP, CH), jnp.float32),
    )(h, av, dv, acc, acc, acc, acc, den, den, b)


def _pool_body(bt_ref, h_ref, wl_ref, bl_ref, o_ref):
    bt = bt_ref[...]
    gid = lax.broadcasted_iota(jnp.int32, (G, NP), 0)
    m = (bt[None, :] == gid).astype(jnp.float32)
    sums = jnp.dot(m, h_ref[...], preferred_element_type=jnp.float32)
    counts = jnp.sum(m, axis=1)
    pooled = sums / jnp.maximum(counts, 1.0)[:, None]
    o_ref[...] = (jnp.dot(pooled, wl_ref[...], preferred_element_type=jnp.float32)
                  + bl_ref[...][None, :])


def _pool(batch_p, h, wl, bl):
    return pl.pallas_call(
        _pool_body,
        out_shape=jax.ShapeDtypeStruct((G, OUT), jnp.float32),
    )(batch_p, h, wl, bl)


def kernel(x, edge_index, batch, W1, a1_src, a1_dst, b1,
           W2, a2_src, a2_dst, b2, Wl, bl):
    xp = jnp.zeros((NP, CH), jnp.float32).at[:N].set(x)
    src = edge_index[0]
    dst = edge_index[1]
    pad = EPAD - E
    srcp = jnp.concatenate([src, jnp.zeros((pad,), jnp.int32)])
    dstp = jnp.concatenate([dst, jnp.full((pad,), N, jnp.int32)])
    eint = jnp.stack([srcp.reshape(-1, EB), dstp.reshape(-1, EB)],
                     axis=1).reshape(-1)
    batch_p = jnp.concatenate([batch, jnp.full((NP - N,), G, jnp.int32)])
    z2 = jnp.zeros((STRIPE, HW), jnp.float32)
    z1 = jnp.zeros((DSTRIPE,), jnp.float32)

    h = xp
    for w, asr, adr, b in ((W1, a1_src, a1_dst, b1), (W2, a2_src, a2_dst, b2)):
        table, hd, av, dv = _prologue(h, w, asr, adr)
        acc, den = _edge_call(table, eint, av, dv, z2, z1)
        h = _combine(hd, av, dv, acc, den, b)
    return _pool(batch_p, h, Wl, bl)


# speedup vs baseline: 1.0201x; 1.0201x over previous
import jax
import jax.numpy as jnp
from jax import lax
from jax.experimental import pallas as pl
from jax.experimental.pallas import tpu as pltpu
from jax.experimental.pallas import tpu_sc as plsc

N = 10000
E = 320000
CH = 128
HW = 64
OUT = 16
G = 64
NEG = 0.2

NP = 10240
NB = 10
RB = NP // NB

NSUB = 16
EB = 128
NSLOT = 3
NBLK = 159
EPT = NBLK * EB
EPAD = EPT * NSUB
STRIPE = 2 * NP // NSUB
DSTRIPE = NP // NSUB


def _prologue_body(x_ref, w_ref, asr_ref, adr_ref, t_ref, h_ref, a_ref, d_ref):
    hf = pl.program_id(0)
    cls = pl.program_id(1)
    h = jnp.dot(x_ref[...], w_ref[...], preferred_element_type=jnp.float32)
    av = jnp.sum(h * asr_ref[...], axis=1)
    dv = jnp.sum(h * adr_ref[...], axis=1)
    scale = jnp.where(cls == 0, 1.0, NEG)
    p = jnp.exp(scale * av)
    ph = p[:, None] * h
    t_ref[...] = jnp.where(hf == 0, ph[:, :HW], ph[:, HW:])
    h_ref[...] = h
    a_ref[...] = av
    d_ref[...] = dv


def _prologue(xp, w, asr, adr):
    return pl.pallas_call(
        _prologue_body,
        grid=(2, 2, NB),
        in_specs=[
            pl.BlockSpec((RB, CH), lambda hf, c, i: (i, 0)),
            pl.BlockSpec((CH, CH), lambda hf, c, i: (0, 0)),
            pl.BlockSpec((1, CH), lambda hf, c, i: (0, 0)),
            pl.BlockSpec((1, CH), lambda hf, c, i: (0, 0)),
        ],
        out_specs=[
            pl.BlockSpec((RB, HW), lambda hf, c, i: ((hf * 2 + c) * NB + i, 0)),
            pl.BlockSpec((RB, CH), lambda hf, c, i: (i, 0)),
            pl.BlockSpec((RB,), lambda hf, c, i: (i,)),
            pl.BlockSpec((RB,), lambda hf, c, i: (i,)),
        ],
        out_shape=[
            jax.ShapeDtypeStruct((4 * NP, HW), jnp.float32),
            jax.ShapeDtypeStruct((NP, CH), jnp.float32),
            jax.ShapeDtypeStruct((NP,), jnp.float32),
            jax.ShapeDtypeStruct((NP,), jnp.float32),
        ],
    )(xp, w, asr, adr)


def _edge_body(t_hbm, ei_hbm, as_hbm, ad_hbm, z2_hbm, z1_hbm,
               acc_hbm, den_hbm,
               as_v, ad_v, ebuf,
               gbuf0, sidx0, didx0, dval0, rows0,
               gbuf1, sidx1, didx1, dval1, rows1,
               gbuf2, sidx2, didx2, dval2, rows2,
               acc_s, den_s,
               semg0, semg1, semg2,
               semr0, semr1, semr2,
               semd0, semd1, semd2):
    c = lax.axis_index("c")
    s = lax.axis_index("s")
    slots = ((gbuf0, sidx0, didx0, dval0, rows0, semg0, semr0, semd0),
             (gbuf1, sidx1, didx1, dval1, rows1, semg1, semr1, semd1),
             (gbuf2, sidx2, didx2, dval2, rows2, semg2, semr2, semd2))

    pltpu.sync_copy(z2_hbm, acc_s.at[pl.ds(s * STRIPE, STRIPE)])
    pltpu.sync_copy(z1_hbm, den_s.at[pl.ds(s * DSTRIPE, DSTRIPE)])
    pltpu.sync_copy(as_hbm, as_v)
    pltpu.sync_copy(ad_hbm, ad_v)
    plsc.subcore_barrier()

    cneg = jnp.broadcast_to(c == 1, (16,))
    trash = jnp.broadcast_to(N + s, (16,))
    goff = jnp.broadcast_to(c * (2 * NP), (16,))
    npvec = jnp.broadcast_to(NP, (16,))
    zvec = jnp.zeros((16,), jnp.int32)
    ascale = jnp.where(c == 1, NEG, 1.0)
    bbase = s * NBLK

    def classify(off, gbuf, sidx, didx, dval):
        def grp(j, carry2):
            s16 = ebuf[pl.ds(off + j * 16, 16)]
            d16 = ebuf[pl.ds(off + EB + j * 16, 16)]
            a16 = plsc.load_gather(as_v, [s16])
            b16 = plsc.load_gather(ad_v, [d16])
            pos = (a16 + b16) > 0.0
            clsoff = jnp.where(pos, zvec, npvec)
            gbuf[pl.ds(j * 16, 16)] = s16 + goff + clsoff
            sidx[pl.ds(j * 16, 16)] = d16 + clsoff
            didx[pl.ds(j * 16, 16)] = jnp.where(pos != cneg, d16, trash)
            dval[pl.ds(j * 16, 16)] = jnp.exp(ascale * a16)
            return carry2

        lax.fori_loop(0, EB // 16, grp, 0)

    def fire_g(slot):
        gbuf, _, _, _, rows, semg, _, _ = slot
        pltpu.async_copy(t_hbm.at[gbuf], rows, semg)

    def fire_s(slot):
        _, sidx, didx, dval, rows, _, semr, semd = slot
        pltpu.async_copy(rows, acc_s.at[sidx], semr, add=True)
        pltpu.async_copy(dval, den_s.at[didx], semd, add=True)

    def wait_g(slot):
        gbuf, _, _, _, rows, semg, _, _ = slot
        pltpu.make_async_copy(t_hbm.at[gbuf], rows, semg).wait()

    def wait_s(slot):
        _, sidx, didx, dval, rows, _, semr, semd = slot
        pltpu.make_async_copy(rows, acc_s.at[sidx], semr).wait()
        pltpu.make_async_copy(dval, den_s.at[didx], semd).wait()

    def quad(i, first):
        base = (bbase + NSLOT * i) * 2 * EB
        pltpu.sync_copy(ei_hbm.at[pl.ds(base, NSLOT * 2 * EB)], ebuf)
        for k in range(NSLOT):
            if not first:
                wait_s(slots[k])
            classify(k * 2 * EB, *slots[k][:4])
            fire_g(slots[k])
        for k in range(NSLOT):
            wait_g(slots[k])
            fire_s(slots[k])

    quad(0, True)
    lax.fori_loop(1, NBLK // NSLOT, lambda i, cy: (quad(i, False), cy)[1], 0)
    for k in range(NSLOT):
        wait_s(slots[k])
    plsc.subcore_barrier()

    pltpu.sync_copy(acc_s.at[pl.ds(s * STRIPE, STRIPE)],
                    acc_hbm.at[pl.ds(c * 2 * NP + s * STRIPE, STRIPE)])
    pltpu.sync_copy(den_s.at[pl.ds(s * DSTRIPE, DSTRIPE)],
                    den_hbm.at[pl.ds(c * NP + s * DSTRIPE, DSTRIPE)])


def _edge_call(*args):
    return pl.kernel(
        _edge_body,
        out_type=[jax.ShapeDtypeStruct((4 * NP, HW), jnp.float32),
                  jax.ShapeDtypeStruct((2 * NP,), jnp.float32)],
        mesh=plsc.VectorSubcoreMesh(core_axis_name="c", subcore_axis_name="s",
                                    num_cores=2, num_subcores=NSUB),
        compiler_params=pltpu.CompilerParams(needs_layout_passes=False,
                                             use_tc_tiling_on_sc=False),
        scratch_types=(
            [pltpu.VMEM((NP,), jnp.float32),
             pltpu.VMEM((NP,), jnp.float32),
             pltpu.VMEM((NSLOT * 2 * EB,), jnp.int32)]
            + NSLOT * [pltpu.VMEM((EB,), jnp.int32),
                       pltpu.VMEM((EB,), jnp.int32),
                       pltpu.VMEM((EB,), jnp.int32),
                       pltpu.VMEM((EB,), jnp.float32),
                       pltpu.VMEM((EB, HW), jnp.float32)]
            + [pltpu.VMEM_SHARED((2 * NP, HW), jnp.float32),
               pltpu.VMEM_SHARED((NP,), jnp.float32)]
            + 3 * NSLOT * [pltpu.SemaphoreType.DMA]
        ),
    )(*args)


def _combine_body(h_ref, a_ref, d_ref, p0_ref, n0_ref, p1_ref, n1_ref,
                  denp_ref, denn_ref, b_ref, o_ref):
    av = a_ref[...]
    dv = d_ref[...]
    q = jnp.exp(dv)
    q2 = jnp.exp(NEG * dv)
    sl = av + dv
    wself = jnp.where(sl > 0.0, jnp.exp(sl), jnp.exp(NEG * sl))
    hv = h_ref[...]
    den = q * denp_ref[...] + q2 * denn_ref[...] + wself
    inv = 1.0 / (den + 1e-16)
    lo = (q[:, None] * p0_ref[...] + q2[:, None] * n0_ref[...]
          + wself[:, None] * hv[:, :HW])
    hi = (q[:, None] * p1_ref[...] + q2[:, None] * n1_ref[...]
          + wself[:, None] * hv[:, HW:])
    bv = b_ref[...]
    o_ref[...] = jnp.maximum(
        jnp.concatenate([lo, hi], axis=1) * inv[:, None] + bv[None, :], 0.0)


def _combine(h, av, dv, acc, den, b):
    return pl.pallas_call(
        _combine_body,
        grid=(NB,),
        in_specs=[
            pl.BlockSpec((RB, CH), lambda i: (i, 0)),
            pl.BlockSpec((RB,), lambda i: (i,)),
            pl.BlockSpec((RB,), lambda i: (i,)),
            pl.BlockSpec((RB, HW), lambda i: (i, 0)),
            pl.BlockSpec((RB, HW), lambda i: (NB + i, 0)),
            pl.BlockSpec((RB, HW), lambda i: (2 * NB + i, 0)),
            pl.BlockSpec((RB, HW), lambda i: (3 * NB + i, 0)),
            pl.BlockSpec((RB,), lambda i: (i,)),
            pl.BlockSpec((RB,), lambda i: (NB + i,)),
            pl.BlockSpec((CH,), lambda i: (0,)),
        ],
        out_specs=pl.BlockSpec((RB, CH), lambda i: (i, 0)),
        out_shape=jax.ShapeDtypeStruct((NP, CH), jnp.float32),
    )(h, av, dv, acc, acc, acc, acc, den, den, b)


def _pool_body(bt_ref, h_ref, wl_ref, bl_ref, o_ref):
    bt = bt_ref[...]
    gid = lax.broadcasted_iota(jnp.int32, (G, NP), 0)
    m = (bt[None, :] == gid).astype(jnp.float32)
    sums = jnp.dot(m, h_ref[...], preferred_element_type=jnp.float32)
    counts = jnp.sum(m, axis=1)
    pooled = sums / jnp.maximum(counts, 1.0)[:, None]
    o_ref[...] = (jnp.dot(pooled, wl_ref[...], preferred_element_type=jnp.float32)
                  + bl_ref[...][None, :])


def _pool(batch_p, h, wl, bl):
    return pl.pallas_call(
        _pool_body,
        out_shape=jax.ShapeDtypeStruct((G, OUT), jnp.float32),
    )(batch_p, h, wl, bl)


def kernel(x, edge_index, batch, W1, a1_src, a1_dst, b1,
           W2, a2_src, a2_dst, b2, Wl, bl):
    xp = jnp.zeros((NP, CH), jnp.float32).at[:N].set(x)
    src = edge_index[0]
    dst = edge_index[1]
    pad = EPAD - E
    srcp = jnp.concatenate([src, jnp.zeros((pad,), jnp.int32)])
    dstp = jnp.concatenate([dst, jnp.full((pad,), N, jnp.int32)])
    eint = jnp.stack([srcp.reshape(-1, EB), dstp.reshape(-1, EB)],
                     axis=1).reshape(-1)
    batch_p = jnp.concatenate([batch, jnp.full((NP - N,), G, jnp.int32)])
    z2 = jnp.zeros((STRIPE, HW), jnp.float32)
    z1 = jnp.zeros((DSTRIPE,), jnp.float32)

    h = xp
    for w, asr, adr, b in ((W1, a1_src, a1_dst, b1), (W2, a2_src, a2_dst, b2)):
        table, hd, av, dv = _prologue(h, w, asr, adr)
        acc, den = _edge_call(table, eint, av, dv, z2, z1)
        h = _combine(hd, av, dv, acc, den, b)
    return _pool(batch_p, h, Wl, bl)


# speedup vs baseline: 1.2516x; 1.2270x over previous
import jax
import jax.numpy as jnp
from jax import lax
from jax.experimental import pallas as pl
from jax.experimental.pallas import tpu as pltpu
from jax.experimental.pallas import tpu_sc as plsc

N = 10000
E = 320000
CH = 128
HW = 64
OUT = 16
G = 64
NEG = 0.2

NP = 10240
NB = 10
RB = NP // NB

NSUB = 16
EB = 128
NBLK = 157
EPT = NBLK * EB
EPAD = EPT * NSUB
STRIPE = 2 * NP // NSUB
DSTRIPE = NP // NSUB


def _prologue_body(x_ref, w_ref, asr_ref, adr_ref, t_ref, h_ref, a_ref, d_ref):
    hf = pl.program_id(1)
    cls = pl.program_id(2)
    h = jnp.dot(x_ref[...], w_ref[...], preferred_element_type=jnp.float32)
    av = jnp.sum(h * asr_ref[...], axis=1)
    dv = jnp.sum(h * adr_ref[...], axis=1)
    scale = jnp.where(cls == 0, 1.0, NEG)
    p = jnp.exp(scale * av)
    ph = p[:, None] * h
    t_ref[...] = jnp.where(hf == 0, ph[:, :HW], ph[:, HW:])
    h_ref[...] = h
    a_ref[...] = av
    d_ref[...] = dv


def _prologue(xp, w, asr, adr):
    return pl.pallas_call(
        _prologue_body,
        grid=(NB, 2, 2),
        in_specs=[
            pl.BlockSpec((RB, CH), lambda i, hf, c: (i, 0)),
            pl.BlockSpec((CH, CH), lambda i, hf, c: (0, 0)),
            pl.BlockSpec((1, CH), lambda i, hf, c: (0, 0)),
            pl.BlockSpec((1, CH), lambda i, hf, c: (0, 0)),
        ],
        out_specs=[
            pl.BlockSpec((RB, HW), lambda i, hf, c: ((hf * 2 + c) * NB + i, 0)),
            pl.BlockSpec((RB, CH), lambda i, hf, c: (i, 0)),
            pl.BlockSpec((RB,), lambda i, hf, c: (i,)),
            pl.BlockSpec((RB,), lambda i, hf, c: (i,)),
        ],
        out_shape=[
            jax.ShapeDtypeStruct((4 * NP, HW), jnp.float32),
            jax.ShapeDtypeStruct((NP, CH), jnp.float32),
            jax.ShapeDtypeStruct((NP,), jnp.float32),
            jax.ShapeDtypeStruct((NP,), jnp.float32),
        ],
    )(xp, w, asr, adr)


def _edge_body(t_hbm, ei_hbm, as_hbm, ad_hbm, z2_hbm, z1_hbm,
               acc_hbm, den_hbm,
               as_v, ad_v, ebuf,
               gbuf0, sidx0, didx0, dval0, rows0,
               gbuf1, sidx1, didx1, dval1, rows1,
               acc_s, den_s, sem0, sem1):
    c = lax.axis_index("c")
    s = lax.axis_index("s")

    pltpu.sync_copy(z2_hbm, acc_s.at[pl.ds(s * STRIPE, STRIPE)])
    pltpu.sync_copy(z1_hbm, den_s.at[pl.ds(s * DSTRIPE, DSTRIPE)])
    pltpu.sync_copy(as_hbm, as_v)
    pltpu.sync_copy(ad_hbm, ad_v)
    plsc.subcore_barrier()

    cneg = jnp.broadcast_to(c == 1, (16,))
    trash = jnp.broadcast_to(N + s, (16,))
    goff = jnp.broadcast_to(c * (2 * NP), (16,))
    npvec = jnp.broadcast_to(NP, (16,))
    zvec = jnp.zeros((16,), jnp.int32)
    ascale = jnp.where(c == 1, NEG, 1.0)
    bbase = s * NBLK

    def classify(off, gbuf, sidx, didx, dval):
        def grp(j, carry2):
            s16 = ebuf[pl.ds(off + j * 16, 16)]
            d16 = ebuf[pl.ds(off + EB + j * 16, 16)]
            a16 = plsc.load_gather(as_v, [s16])
            b16 = plsc.load_gather(ad_v, [d16])
            pos = (a16 + b16) > 0.0
            clsoff = jnp.where(pos, zvec, npvec)
            gbuf[pl.ds(j * 16, 16)] = s16 + goff + clsoff
            sidx[pl.ds(j * 16, 16)] = d16 + clsoff
            didx[pl.ds(j * 16, 16)] = jnp.where(pos != cneg, d16, trash)
            dval[pl.ds(j * 16, 16)] = jnp.exp(ascale * a16)
            return carry2

        lax.fori_loop(0, EB // 16, grp, 0)

    def fire(gbuf, rows, sem):
        pltpu.async_copy(t_hbm.at[gbuf], rows, sem)

    def scatter(rows, sidx, didx, dval):
        pltpu.sync_copy(rows, acc_s.at[sidx], add=True)
        pltpu.sync_copy(dval, den_s.at[didx], add=True)

    pltpu.sync_copy(ei_hbm.at[pl.ds(bbase * 2 * EB, 2 * EB)],
                    ebuf.at[pl.ds(0, 2 * EB)])
    classify(0, gbuf0, sidx0, didx0, dval0)
    fire(gbuf0, rows0, sem0)

    def pair(g, carry):
        b1 = 2 * g + 1
        pltpu.sync_copy(ei_hbm.at[pl.ds((bbase + b1) * 2 * EB, 4 * EB)], ebuf)
        classify(0, gbuf1, sidx1, didx1, dval1)
        fire(gbuf1, rows1, sem1)
        pltpu.make_async_copy(t_hbm.at[gbuf0], rows0, sem0).wait()
        scatter(rows0, sidx0, didx0, dval0)
        classify(2 * EB, gbuf0, sidx0, didx0, dval0)
        fire(gbuf0, rows0, sem0)
        pltpu.make_async_copy(t_hbm.at[gbuf1], rows1, sem1).wait()
        scatter(rows1, sidx1, didx1, dval1)
        return carry

    lax.fori_loop(0, (NBLK - 1) // 2, pair, 0)
    pltpu.make_async_copy(t_hbm.at[gbuf0], rows0, sem0).wait()
    scatter(rows0, sidx0, didx0, dval0)
    plsc.subcore_barrier()

    pltpu.sync_copy(acc_s.at[pl.ds(s * STRIPE, STRIPE)],
                    acc_hbm.at[pl.ds(c * 2 * NP + s * STRIPE, STRIPE)])
    pltpu.sync_copy(den_s.at[pl.ds(s * DSTRIPE, DSTRIPE)],
                    den_hbm.at[pl.ds(c * NP + s * DSTRIPE, DSTRIPE)])


def _edge_call(*args):
    return pl.kernel(
        _edge_body,
        out_type=[jax.ShapeDtypeStruct((4 * NP, HW), jnp.float32),
                  jax.ShapeDtypeStruct((2 * NP,), jnp.float32)],
        mesh=plsc.VectorSubcoreMesh(core_axis_name="c", subcore_axis_name="s",
                                    num_cores=2, num_subcores=NSUB),
        compiler_params=pltpu.CompilerParams(needs_layout_passes=False,
                                             use_tc_tiling_on_sc=False),
        scratch_types=(
            [pltpu.VMEM((NP,), jnp.float32),
             pltpu.VMEM((NP,), jnp.float32),
             pltpu.VMEM((4 * EB,), jnp.int32)]
            + 2 * [pltpu.VMEM((EB,), jnp.int32),
                   pltpu.VMEM((EB,), jnp.int32),
                   pltpu.VMEM((EB,), jnp.int32),
                   pltpu.VMEM((EB,), jnp.float32),
                   pltpu.VMEM((EB, HW), jnp.float32)]
            + [pltpu.VMEM_SHARED((2 * NP, HW), jnp.float32),
               pltpu.VMEM_SHARED((NP,), jnp.float32),
               pltpu.SemaphoreType.DMA,
               pltpu.SemaphoreType.DMA]
        ),
    )(*args)


def _combine_body(h_ref, a_ref, d_ref, p0_ref, n0_ref, p1_ref, n1_ref,
                  denp_ref, denn_ref, b_ref, o_ref):
    av = a_ref[...]
    dv = d_ref[...]
    q = jnp.exp(dv)
    q2 = jnp.exp(NEG * dv)
    sl = av + dv
    wself = jnp.where(sl > 0.0, jnp.exp(sl), jnp.exp(NEG * sl))
    hv = h_ref[...]
    den = q * denp_ref[...] + q2 * denn_ref[...] + wself
    inv = 1.0 / (den + 1e-16)
    lo = (q[:, None] * p0_ref[...] + q2[:, None] * n0_ref[...]
          + wself[:, None] * hv[:, :HW])
    hi = (q[:, None] * p1_ref[...] + q2[:, None] * n1_ref[...]
          + wself[:, None] * hv[:, HW:])
    bv = b_ref[...]
    o_ref[...] = jnp.maximum(
        jnp.concatenate([lo, hi], axis=1) * inv[:, None] + bv[None, :], 0.0)


def _combine(h, av, dv, acc, den, b):
    return pl.pallas_call(
        _combine_body,
        grid=(NB,),
        in_specs=[
            pl.BlockSpec((RB, CH), lambda i: (i, 0)),
            pl.BlockSpec((RB,), lambda i: (i,)),
            pl.BlockSpec((RB,), lambda i: (i,)),
            pl.BlockSpec((RB, HW), lambda i: (i, 0)),
            pl.BlockSpec((RB, HW), lambda i: (NB + i, 0)),
            pl.BlockSpec((RB, HW), lambda i: (2 * NB + i, 0)),
            pl.BlockSpec((RB, HW), lambda i: (3 * NB + i, 0)),
            pl.BlockSpec((RB,), lambda i: (i,)),
            pl.BlockSpec((RB,), lambda i: (NB + i,)),
            pl.BlockSpec((CH,), lambda i: (0,)),
        ],
        out_specs=pl.BlockSpec((RB, CH), lambda i: (i, 0)),
        out_shape=jax.ShapeDtypeStruct((NP, CH), jnp.float32),
    )(h, av, dv, acc, acc, acc, acc, den, den, b)


def _pool_body(bt_ref, h_ref, wl_ref, bl_ref, o_ref):
    bt = bt_ref[...]
    gid = lax.broadcasted_iota(jnp.int32, (G, NP), 0)
    m = (bt[None, :] == gid).astype(jnp.float32)
    sums = jnp.dot(m, h_ref[...], preferred_element_type=jnp.float32)
    counts = jnp.sum(m, axis=1)
    pooled = sums / jnp.maximum(counts, 1.0)[:, None]
    o_ref[...] = (jnp.dot(pooled, wl_ref[...], preferred_element_type=jnp.float32)
                  + bl_ref[...][None, :])


def _pool(batch_p, h, wl, bl):
    return pl.pallas_call(
        _pool_body,
        out_shape=jax.ShapeDtypeStruct((G, OUT), jnp.float32),
    )(batch_p, h, wl, bl)


def kernel(x, edge_index, batch, W1, a1_src, a1_dst, b1,
           W2, a2_src, a2_dst, b2, Wl, bl):
    xp = jnp.zeros((NP, CH), jnp.float32).at[:N].set(x)
    src = edge_index[0]
    dst = edge_index[1]
    pad = EPAD - E
    srcp = jnp.concatenate([src, jnp.zeros((pad,), jnp.int32)])
    dstp = jnp.concatenate([dst, jnp.full((pad,), N, jnp.int32)])
    eint = jnp.stack([srcp.reshape(-1, EB), dstp.reshape(-1, EB)],
                     axis=1).reshape(-1)
    batch_p = jnp.concatenate([batch, jnp.full((NP - N,), G, jnp.int32)])
    z2 = jnp.zeros((STRIPE, HW), jnp.float32)
    z1 = jnp.zeros((DSTRIPE,), jnp.float32)

    h = xp
    for w, asr, adr, b in ((W1, a1_src, a1_dst, b1), (W2, a2_src, a2_dst, b2)):
        table, hd, av, dv = _prologue(h, w, asr, adr)
        acc, den = _edge_call(table, eint, av, dv, z2, z1)
        h = _combine(hd, av, dv, acc, den, b)
    return _pool(batch_p, h, Wl, bl)
